# baseline (speedup 1.0000x reference)
import jax
import jax.numpy as jnp
from jax.experimental import pallas as pl
from jax.experimental.pallas import tpu as pltpu

_TI = 64
_CC = 8
_HP = jax.lax.Precision.HIGHEST


def _proj_body(x_ref, wlT_ref, bl_ref, wlbT_ref, blb_ref, npos_ref,
               hl_ref, hlbT_ref, alrow_ref):
    cout = wlT_ref.shape[1]
    npos = npos_ref[0, 0]
    x = x_ref[...]
    hl_ref[...] = jnp.dot(x, wlT_ref[...], precision=_HP) + bl_ref[...]
    hlb = jnp.dot(x, wlbT_ref[...], precision=_HP) + blb_ref[...]
    hlbT = hlb.T
    hlbT_ref[...] = hlbT
    sgn_s = jnp.where(
        jax.lax.broadcasted_iota(jnp.int32, (cout, 1), 0) < npos, 1.0, -1.0)
    alrow_ref[...] = 1.5 * jnp.sum(sgn_s * hlbT, axis=0, keepdims=True)


def _proj(x, wlT, bl, wlbT, blb, npos):
    n, cin = x.shape
    cout = wlT.shape[1]
    return pl.pallas_call(
        _proj_body,
        in_specs=[
            pl.BlockSpec((n, cin), lambda: (0, 0)),
            pl.BlockSpec((cin, cout), lambda: (0, 0)),
            pl.BlockSpec((1, cout), lambda: (0, 0)),
            pl.BlockSpec((cin, cout), lambda: (0, 0)),
            pl.BlockSpec((1, cout), lambda: (0, 0)),
            pl.BlockSpec(memory_space=pltpu.SMEM),
        ],
        out_specs=[
            pl.BlockSpec((n, cout), lambda: (0, 0)),
            pl.BlockSpec((cout, n), lambda: (0, 0)),
            pl.BlockSpec((1, n), lambda: (0, 0)),
        ],
        out_shape=[
            jax.ShapeDtypeStruct((n, cout), jnp.float32),
            jax.ShapeDtypeStruct((cout, n), jnp.float32),
            jax.ShapeDtypeStruct((1, n), jnp.float32),
        ],
    )(x, wlT, bl, wlbT, blb, npos)


def _gat_body(x_ref, maskT_ref, wlbT_ref, blb_ref, wrbT_ref, brb_ref,
              npos_ref, bias_ref, hl_ref, hlbT_ref, alrow_ref,
              out_ref, s_ref):
    it = pl.program_id(0)
    n = maskT_ref.shape[1]
    cout = wlbT_ref.shape[1]
    nchunks = cout // _CC
    npos = npos_ref[0, 0]

    x_t = x_ref[...]
    hrb_t = jnp.dot(x_t, wrbT_ref[...], precision=_HP) + brb_ref[...]
    hlb_t = jnp.dot(x_t, wlbT_ref[...], precision=_HP) + blb_ref[...]

    sgn_l = jnp.where(
        jax.lax.broadcasted_iota(jnp.int32, (1, cout), 1) < npos, 1.0, -1.0)

    tb_d = hrb_t + hlb_t
    d = (1.5 * jnp.sum(sgn_l * tb_d, axis=1, keepdims=True)
         + jnp.sum(sgn_l * jnp.abs(tb_d), axis=1, keepdims=True))

    ar = 1.5 * jnp.sum(sgn_l * hrb_t, axis=1, keepdims=True)
    s_ref[...] = ar + alrow_ref[...]

    sub_iota8 = jax.lax.broadcasted_iota(jnp.int32, (cout, _CC), 0)
    lane_iota8 = jax.lax.broadcasted_iota(jnp.int32, (cout, _CC), 1)

    def chunk_abs(k):
        c0 = k * _CC
        oh = (sub_iota8 == c0 + lane_iota8).astype(jnp.float32)
        cols = jnp.dot(hrb_t, oh, precision=_HP)
        terms = []
        for u in range(_CC):
            col = cols[:, u:u + 1]
            row = hlbT_ref[pl.ds(c0 + u, 1), :]
            terms.append(jnp.abs(col + row))
        return terms

    def tree_sum(ts):
        while len(ts) > 1:
            ts = [a + b for a, b in zip(ts[::2], ts[1::2])]
        return ts[0]

    kpos = npos // _CC

    def body_pos(k, carry):
        s_ref[...] += tree_sum(chunk_abs(k))
        return carry

    def body_neg(k, carry):
        s_ref[...] -= tree_sum(chunk_abs(k))
        return carry

    jax.lax.fori_loop(0, kpos, body_pos, 0)

    @pl.when(kpos < nchunks)
    def _straddle():
        c0 = kpos * _CC
        ts = chunk_abs(kpos)
        acc = None
        for u, t in enumerate(ts):
            sg = jnp.where(c0 + u < npos, jnp.float32(1.0), jnp.float32(-1.0))
            term = sg * t
            acc = term if acc is None else acc + term
        s_ref[...] += acc

    jax.lax.fori_loop(kpos + 1, nchunks, body_neg, 0)

    S = s_ref[...]
    mask = maskT_ref[...] > 0
    mx = jnp.max(jnp.where(mask, S, -jnp.inf), axis=1, keepdims=True)
    mx = jnp.maximum(mx, d)
    P = jnp.where(mask, jnp.exp(S - mx), 0.0)
    p_self = jnp.exp(d - mx)
    denom = jnp.sum(P, axis=1, keepdims=True) + p_self + 1e-16
    hl = hl_ref[...]
    hl_t = hl_ref[pl.ds(it * _TI, _TI), :]
    num = jnp.dot(P, hl, precision=_HP) + p_self * hl_t
    out = num / denom + bias_ref[...]
    out_ref[...] = jnp.maximum(out, 0.0)


def _gat_layer(x, maskT, p):
    n, cin = x.shape
    cout = p["Wl"].shape[0]
    att = p["att"]
    pos = att >= 0
    order = jnp.argsort(jnp.logical_not(pos), stable=True)
    npos = jnp.sum(pos).astype(jnp.int32).reshape(1, 1)
    sa = (0.4 * jnp.abs(att))[order]
    wlT = p["Wl"].T
    bl = p["bl"].reshape(1, cout)
    wlbT = (p["Wl"][order] * sa[:, None]).T
    blb = (p["bl"][order] * sa).reshape(1, cout)
    wrbT = (p["Wr"][order] * sa[:, None]).T
    brb = (p["br"][order] * sa).reshape(1, cout)
    bias = p["bias"].reshape(1, cout)
    hl, hlbT, alrow = _proj(x, wlT, bl, wlbT, blb, npos)
    return pl.pallas_call(
        _gat_body,
        grid=(n // _TI,),
        in_specs=[
            pl.BlockSpec((_TI, cin), lambda i: (i, 0)),
            pl.BlockSpec((_TI, n), lambda i: (i, 0)),
            pl.BlockSpec((cin, cout), lambda i: (0, 0)),
            pl.BlockSpec((1, cout), lambda i: (0, 0)),
            pl.BlockSpec((cin, cout), lambda i: (0, 0)),
            pl.BlockSpec((1, cout), lambda i: (0, 0)),
            pl.BlockSpec(memory_space=pltpu.SMEM),
            pl.BlockSpec((1, cout), lambda i: (0, 0)),
            pl.BlockSpec((n, cout), lambda i: (0, 0)),
            pl.BlockSpec((cout, n), lambda i: (0, 0)),
            pl.BlockSpec((1, n), lambda i: (0, 0)),
        ],
        out_specs=pl.BlockSpec((_TI, cout), lambda i: (i, 0)),
        out_shape=jax.ShapeDtypeStruct((n, cout), jnp.float32),
        scratch_shapes=[pltpu.VMEM((_TI, n), jnp.float32)],
        compiler_params=pltpu.CompilerParams(
            dimension_semantics=("parallel",)),
    )(x, maskT, wlbT, blb, wrbT, brb, npos, bias, hl, hlbT, alrow)


def _recon_body(re_ref, out_ref):
    it = pl.program_id(0)
    re = re_ref[...]
    re_t = re_ref[pl.ds(it * _TI, _TI), :]
    logits = jnp.dot(re_t, re.T, precision=_HP)
    out_ref[...] = jax.nn.sigmoid(logits)


def _recon(re):
    n, c = re.shape
    return pl.pallas_call(
        _recon_body,
        grid=(n // _TI,),
        in_specs=[pl.BlockSpec((n, c), lambda i: (0, 0))],
        out_specs=pl.BlockSpec((_TI, n), lambda i: (i, 0)),
        out_shape=jax.ShapeDtypeStruct((n, n), jnp.float32),
        compiler_params=pltpu.CompilerParams(
            dimension_semantics=("parallel",)),
    )(re)


def kernel(x, edge_index, params):
    maskT = (edge_index.T != 0).astype(jnp.float32)
    x1 = _gat_layer(x, maskT, params["conv1"])
    z = _gat_layer(x1, maskT, params["conv2"])
    re = _gat_layer(z, maskT, params["edge_dec"])
    recon = _recon(re)
    xd = _gat_layer(z, maskT, params["x_dec1"])
    xr = _gat_layer(xd, maskT, params["x_dec2"])
    return recon, xr, z


# speedup vs baseline: 1.4006x; 1.4006x over previous
import jax
import jax.numpy as jnp
from jax.experimental import pallas as pl
from jax.experimental.pallas import tpu as pltpu

_TI = 256
_CC = 4
_HP = jax.lax.Precision.HIGHEST


def _gat_body(x_ref, maskT_ref, wlT_ref, bl_ref, wlbT_ref, blb_ref,
              wrbT_ref, brb_ref, npos_ref, bias_ref, out_ref, s_ref,
              hlbT_ref):
    it = pl.program_id(0)
    n = x_ref.shape[0]
    cout = wlT_ref.shape[1]
    nchunks = cout // _CC
    npos = npos_ref[0, 0]

    x = x_ref[...]
    x_t = x_ref[pl.ds(it * _TI, _TI), :]
    hl = jnp.dot(x, wlT_ref[...], precision=_HP) + bl_ref[...]
    hl_t = jnp.dot(x_t, wlT_ref[...], precision=_HP) + bl_ref[...]
    hlbT = (jnp.dot(x, wlbT_ref[...], precision=_HP) + blb_ref[...]).T
    hlbT_ref[...] = hlbT
    hrb_t = jnp.dot(x_t, wrbT_ref[...], precision=_HP) + brb_ref[...]
    hlb_t = jnp.dot(x_t, wlbT_ref[...], precision=_HP) + blb_ref[...]

    sgn_l = jnp.where(
        jax.lax.broadcasted_iota(jnp.int32, (1, cout), 1) < npos, 1.0, -1.0)
    sgn_s = jnp.where(
        jax.lax.broadcasted_iota(jnp.int32, (cout, 1), 0) < npos, 1.0, -1.0)

    tb_d = hrb_t + hlb_t
    d = (1.5 * jnp.sum(sgn_l * tb_d, axis=1, keepdims=True)
         + jnp.sum(sgn_l * jnp.abs(tb_d), axis=1, keepdims=True))

    ar = 1.5 * jnp.sum(sgn_l * hrb_t, axis=1, keepdims=True)
    al_row = 1.5 * jnp.sum(sgn_s * hlbT, axis=0, keepdims=True)
    s_ref[...] = ar + al_row

    lane_iota = jax.lax.broadcasted_iota(jnp.int32, (_TI, cout), 1)

    def chunk_abs(k):
        c0 = k * _CC
        terms = []
        for u in range(_CC):
            c = c0 + u
            col = jnp.sum(jnp.where(lane_iota == c, hrb_t, 0.0), axis=1,
                          keepdims=True)
            row = hlbT_ref[pl.ds(c, 1), :]
            terms.append(jnp.abs(col + row))
        return terms

    def tree_sum(ts):
        while len(ts) > 1:
            ts = [a + b for a, b in zip(ts[::2], ts[1::2])]
        return ts[0]

    kpos = npos // _CC

    def body_pos(k, carry):
        s_ref[...] += tree_sum(chunk_abs(k))
        return carry

    def body_neg(k, carry):
        s_ref[...] -= tree_sum(chunk_abs(k))
        return carry

    jax.lax.fori_loop(0, kpos, body_pos, 0)

    @pl.when(kpos < nchunks)
    def _straddle():
        c0 = kpos * _CC
        ts = chunk_abs(kpos)
        acc = None
        for u, t in enumerate(ts):
            sg = jnp.where(c0 + u < npos, jnp.float32(1.0), jnp.float32(-1.0))
            term = sg * t
            acc = term if acc is None else acc + term
        s_ref[...] += acc

    jax.lax.fori_loop(kpos + 1, nchunks, body_neg, 0)

    S = s_ref[...]
    mask = maskT_ref[...] > 0
    mx = jnp.max(jnp.where(mask, S, -jnp.inf), axis=1, keepdims=True)
    mx = jnp.maximum(mx, d)
    P = jnp.where(mask, jnp.exp(S - mx), 0.0)
    p_self = jnp.exp(d - mx)
    denom = jnp.sum(P, axis=1, keepdims=True) + p_self + 1e-16
    num = jnp.dot(P, hl, precision=_HP) + p_self * hl_t
    out = num / denom + bias_ref[...]
    out_ref[...] = jnp.maximum(out, 0.0)


def _gat_layer(x, maskT, p):
    n, cin = x.shape
    cout = p["Wl"].shape[0]
    att = p["att"]
    pos = att >= 0
    order = jnp.argsort(jnp.logical_not(pos), stable=True)
    npos = jnp.sum(pos).astype(jnp.int32).reshape(1, 1)
    sa = (0.4 * jnp.abs(att))[order]
    wlT = p["Wl"].T
    bl = p["bl"].reshape(1, cout)
    wlbT = (p["Wl"][order] * sa[:, None]).T
    blb = (p["bl"][order] * sa).reshape(1, cout)
    wrbT = (p["Wr"][order] * sa[:, None]).T
    brb = (p["br"][order] * sa).reshape(1, cout)
    bias = p["bias"].reshape(1, cout)
    return pl.pallas_call(
        _gat_body,
        grid=(n // _TI,),
        in_specs=[
            pl.BlockSpec((n, cin), lambda i: (0, 0)),
            pl.BlockSpec((_TI, n), lambda i: (i, 0)),
            pl.BlockSpec((cin, cout), lambda i: (0, 0)),
            pl.BlockSpec((1, cout), lambda i: (0, 0)),
            pl.BlockSpec((cin, cout), lambda i: (0, 0)),
            pl.BlockSpec((1, cout), lambda i: (0, 0)),
            pl.BlockSpec((cin, cout), lambda i: (0, 0)),
            pl.BlockSpec((1, cout), lambda i: (0, 0)),
            pl.BlockSpec(memory_space=pltpu.SMEM),
            pl.BlockSpec((1, cout), lambda i: (0, 0)),
        ],
        out_specs=pl.BlockSpec((_TI, cout), lambda i: (i, 0)),
        out_shape=jax.ShapeDtypeStruct((n, cout), jnp.float32),
        scratch_shapes=[pltpu.VMEM((_TI, n), jnp.float32),
                        pltpu.VMEM((cout, n), jnp.float32)],
        compiler_params=pltpu.CompilerParams(
            dimension_semantics=("parallel",)),
    )(x, maskT, wlT, bl, wlbT, blb, wrbT, brb, npos, bias)


def _recon_body(re_ref, out_ref):
    it = pl.program_id(0)
    re = re_ref[...]
    re_t = re_ref[pl.ds(it * _TI, _TI), :]
    logits = jnp.dot(re_t, re.T, precision=_HP)
    out_ref[...] = jax.nn.sigmoid(logits)


def _recon(re):
    n, c = re.shape
    return pl.pallas_call(
        _recon_body,
        grid=(n // _TI,),
        in_specs=[pl.BlockSpec((n, c), lambda i: (0, 0))],
        out_specs=pl.BlockSpec((_TI, n), lambda i: (i, 0)),
        out_shape=jax.ShapeDtypeStruct((n, n), jnp.float32),
        compiler_params=pltpu.CompilerParams(
            dimension_semantics=("parallel",)),
    )(re)


def kernel(x, edge_index, params):
    maskT = (edge_index.T != 0).astype(jnp.float32)
    x1 = _gat_layer(x, maskT, params["conv1"])
    z = _gat_layer(x1, maskT, params["conv2"])
    re = _gat_layer(z, maskT, params["edge_dec"])
    recon = _recon(re)
    xd = _gat_layer(z, maskT, params["x_dec1"])
    xr = _gat_layer(xd, maskT, params["x_dec2"])
    return recon, xr, z
